# baseline (speedup 1.0000x reference)
import jax
import jax.numpy as jnp
from jax import lax
from jax.experimental import pallas as pl
from jax.experimental.pallas import tpu as pltpu
from jax.experimental.pallas import tpu_sc as plsc

N_NODES = 10000
N_EDGES = 20000
C_DIM = 1280
LANES = 128
NCH = C_DIM // LANES
NC = 2
NS = 16
CH_PER_CORE = NCH // NC

E_PAD = 20480
N_PAD = 10240
EB_TC = 512
B = 40
ET = E_PAD // NS
NB = ET // B
ROWS_PER_TILE = N_PAD // NS


def _film_body(pose_ref, w1_ref, b1_ref, w2g_ref, b2g_ref, w2b_ref, b2b_ref,
               gamma_ref, beta_ref):
    dn = (((1,), (1,)), ((), ()))
    h = lax.dot_general(pose_ref[...], w1_ref[...], dn,
                        preferred_element_type=jnp.float32)
    h = jnp.maximum(h + b1_ref[...], 0.0)
    g = lax.dot_general(h, w2g_ref[...], dn, preferred_element_type=jnp.float32)
    gamma_ref[...] = jax.nn.sigmoid(g + b2g_ref[...])
    bb = lax.dot_general(h, w2b_ref[...], dn, preferred_element_type=jnp.float32)
    beta_ref[...] = jax.nn.sigmoid(bb + b2b_ref[...])


def _film_params(pose_p, W1, b1, W2g, b2g, W2b, b2b):
    n_blk = E_PAD // EB_TC
    full = lambda i: (0, 0)
    return pl.pallas_call(
        _film_body,
        grid=(n_blk,),
        in_specs=[
            pl.BlockSpec((EB_TC, 16), lambda i: (i, 0)),
            pl.BlockSpec((C_DIM, 16), full),
            pl.BlockSpec((1, C_DIM), full),
            pl.BlockSpec((C_DIM, C_DIM), full),
            pl.BlockSpec((1, C_DIM), full),
            pl.BlockSpec((C_DIM, C_DIM), full),
            pl.BlockSpec((1, C_DIM), full),
        ],
        out_specs=[
            pl.BlockSpec((EB_TC, C_DIM), lambda i: (i, 0)),
            pl.BlockSpec((EB_TC, C_DIM), lambda i: (i, 0)),
        ],
        out_shape=[
            jax.ShapeDtypeStruct((E_PAD, C_DIM), jnp.float32),
            jax.ShapeDtypeStruct((E_PAD, C_DIM), jnp.float32),
        ],
    )(pose_p, W1, b1, W2g, b2g, W2b, b2b)


def _sc_body(xt_hbm, gamma_hbm, beta_hbm, src_hbm, dst_hbm, z_hbm,
             msum_hbm, cntp_hbm,
             src_v, dst_v, srca_v,
             xb0, xb1, gb0, gb1, bb0, bb1, pb0, pb1,
             isem0, isem1, osem0, osem1, acc_sh):
    cid = lax.axis_index("c")
    sid = lax.axis_index("s")
    e0 = sid * ET
    row0 = sid * ROWS_PER_TILE
    xbuf = (xb0, xb1)
    gbuf = (gb0, gb1)
    bbuf = (bb0, bb1)
    pbuf = (pb0, pb1)
    isem = (isem0, isem1)
    osem = (osem0, osem1)

    pltpu.sync_copy(src_hbm.at[pl.ds(e0, ET)], src_v)
    for j in range(NB):
        pltpu.sync_copy(dst_hbm.at[pl.ds(e0 + j * B, B)], dst_v.at[j])

    def zero_acc(acc_sh):
        pltpu.sync_copy(z_hbm, acc_sh.at[pl.ds(row0, ROWS_PER_TILE)])

    def issue_inputs(b, s, c):
        pltpu.async_copy(xt_hbm.at[srca_v.at[pl.ds(b * B, B)]],
                         xbuf[s], isem[s])
        pltpu.async_copy(gamma_hbm.at[pl.ds(e0 + b * B, B),
                                      pl.ds(c * LANES, LANES)],
                         gbuf[s], isem[s])
        pltpu.async_copy(beta_hbm.at[pl.ds(e0 + b * B, B),
                                     pl.ds(c * LANES, LANES)],
                         bbuf[s], isem[s])

    def wait_inputs(b, s, c):
        pltpu.make_async_copy(xt_hbm.at[srca_v.at[pl.ds(b * B, B)]],
                              xbuf[s], isem[s]).wait()
        pltpu.make_async_copy(gamma_hbm.at[pl.ds(e0 + b * B, B),
                                           pl.ds(c * LANES, LANES)],
                              gbuf[s], isem[s]).wait()
        pltpu.make_async_copy(beta_hbm.at[pl.ds(e0 + b * B, B),
                                          pl.ds(c * LANES, LANES)],
                              bbuf[s], isem[s]).wait()

    def adjust(c):
        def adj(r, _):
            sl = pl.ds(r * 16, 16)
            srca_v[sl] = src_v[sl] + c
            return 0
        lax.fori_loop(0, ET // 16, adj, 0)

    def run():
        zero_acc(acc_sh)
        c0 = cid * CH_PER_CORE
        adjust(c0)
        issue_inputs(0, 0, c0)
        issue_inputs(1, 1, c0)
        plsc.subcore_barrier()

        for j in range(CH_PER_CORE):
            c = cid * CH_PER_CORE + j

            def step(i, _):
                for s in (0, 1):
                    b = 2 * i + s
                    wait_inputs(b, s, c)

                    @pl.when(i > 0)
                    def _():
                        pltpu.make_async_copy(
                            pbuf[s], acc_sh.at[dst_v.at[b]], osem[s]).wait()

                    def mul_row(r, _):
                        for k in range(LANES // 16):
                            sl = pl.ds(k * 16, 16)
                            pbuf[s][r, sl] = (xbuf[s][r, sl] * gbuf[s][r, sl]
                                              + bbuf[s][r, sl])
                        return 0
                    lax.fori_loop(0, B, mul_row, 0)

                    @pl.when(b + 2 < NB)
                    def _():
                        issue_inputs(b + 2, s, c)

                    pltpu.async_copy(pbuf[s], acc_sh.at[dst_v.at[b]],
                                     osem[s], add=True)
                return 0
            lax.fori_loop(0, NB // 2, step, 0)
            for s in (0, 1):
                pltpu.make_async_copy(pbuf[s], acc_sh.at[dst_v.at[0]],
                                      osem[s]).wait()

            if j != CH_PER_CORE - 1:
                adjust(c + 1)
                issue_inputs(0, 0, c + 1)
                issue_inputs(1, 1, c + 1)

            plsc.subcore_barrier()
            pltpu.sync_copy(acc_sh.at[pl.ds(row0, ROWS_PER_TILE)],
                            msum_hbm.at[pl.ds(c * N_PAD + row0,
                                              ROWS_PER_TILE)])
            zero_acc(acc_sh)
            plsc.subcore_barrier()

        @pl.when(cid == 0)
        def _():
            ones16 = jnp.ones((16,), jnp.float32)

            def orow(r, _):
                for k in range(LANES // 16):
                    pb0[r, pl.ds(k * 16, 16)] = ones16
                return 0
            lax.fori_loop(0, B, orow, 0)
            for b in range(NB):
                pltpu.sync_copy(pb0, acc_sh.at[dst_v.at[b]], add=True)
            plsc.subcore_barrier()
            pltpu.sync_copy(acc_sh.at[pl.ds(row0, ROWS_PER_TILE)],
                            cntp_hbm.at[pl.ds(row0, ROWS_PER_TILE)])

    run()


def _sc_aggregate(x_r, gamma, beta, src_p, dst_p, z):
    mesh = plsc.VectorSubcoreMesh(core_axis_name="c", subcore_axis_name="s")
    fbuf = pltpu.VMEM((B, LANES), jnp.float32)
    return pl.kernel(
        _sc_body,
        out_type=[
            jax.ShapeDtypeStruct((NCH * N_PAD, LANES), jnp.float32),
            jax.ShapeDtypeStruct((N_PAD, LANES), jnp.float32),
        ],
        mesh=mesh,
        compiler_params=pltpu.CompilerParams(use_tc_tiling_on_sc=True),
        scratch_types=[
            pltpu.VMEM((ET,), jnp.int32),
            pltpu.VMEM((NB, B), jnp.int32),
            pltpu.VMEM((ET,), jnp.int32),
            fbuf, fbuf, fbuf, fbuf, fbuf, fbuf, fbuf, fbuf,
            pltpu.SemaphoreType.DMA,
            pltpu.SemaphoreType.DMA,
            pltpu.SemaphoreType.DMA,
            pltpu.SemaphoreType.DMA,
            pltpu.VMEM_SHARED((N_PAD, LANES), jnp.float32),
        ],
    )(x_r, gamma, beta, src_p, dst_p, z)


def _div_body(msum_ref, cntp_ref, out_ref):
    cnt = cntp_ref[:, 0]
    inv = 1.0 / jnp.maximum(cnt, 1.0)
    out_ref[...] = msum_ref[...] * inv[:, None]


def _mean_divide(msum, cntp):
    nblk = 2048
    grid = (N_PAD // nblk, NCH)
    return pl.pallas_call(
        _div_body,
        grid=grid,
        in_specs=[
            pl.BlockSpec((nblk, LANES),
                         lambda i, c: (c * (N_PAD // nblk) + i, 0)),
            pl.BlockSpec((nblk, LANES), lambda i, c: (i, 0)),
        ],
        out_specs=pl.BlockSpec((nblk, LANES), lambda i, c: (i, c)),
        out_shape=jax.ShapeDtypeStruct((N_PAD, C_DIM), jnp.float32),
    )(msum, cntp)


@jax.jit
def kernel(x, pose, W1, b1, W2, b2, edge_index):
    x_r = x.reshape(N_NODES * NCH, LANES)

    src = edge_index[0]
    dst = edge_index[1]
    pad = E_PAD - N_EDGES
    src_p = jnp.concatenate([src * NCH, jnp.zeros((pad,), jnp.int32)])
    dst_p = jnp.concatenate([dst, jnp.full((pad,), N_NODES, jnp.int32)])
    pose_p = jnp.zeros((E_PAD, 16), jnp.float32).at[:N_EDGES, :9].set(pose)

    W1p = jnp.zeros((C_DIM, 16), jnp.float32).at[:, :9].set(W1)
    W2g, W2b = W2[0::2], W2[1::2]
    b2g, b2b = b2[0::2], b2[1::2]

    gamma, beta = _film_params(pose_p, W1p, b1.reshape(1, C_DIM),
                               W2g, b2g.reshape(1, C_DIM),
                               W2b, b2b.reshape(1, C_DIM))

    z = jnp.zeros((ROWS_PER_TILE, LANES), jnp.float32)
    msum, cntp = _sc_aggregate(x_r, gamma, beta, src_p, dst_p, z)

    out = _mean_divide(msum, cntp)
    return out[:N_NODES].reshape(N_NODES, C_DIM, 1, 1)


# speedup vs baseline: 1.0098x; 1.0098x over previous
import jax
import jax.numpy as jnp
from jax import lax
from jax.experimental import pallas as pl
from jax.experimental.pallas import tpu as pltpu
from jax.experimental.pallas import tpu_sc as plsc

N_NODES = 10000
N_EDGES = 20000
C_DIM = 1280
LANES = 128
NCH = C_DIM // LANES
NC = 2
NS = 16
CH_PER_CORE = NCH // NC

E_PAD = 20480
N_PAD = 10240
EB_TC = 256
B = 40
ET = E_PAD // NS
NB = ET // B
ROWS_PER_TILE = N_PAD // NS


def _film_body(pose_ref, w1_ref, b1_ref, w2g_ref, b2g_ref, w2b_ref, b2b_ref,
               gamma_ref, beta_ref):
    dn = (((1,), (1,)), ((), ()))
    h = lax.dot_general(pose_ref[...], w1_ref[...], dn,
                        preferred_element_type=jnp.float32)
    h = jnp.maximum(h + b1_ref[...], 0.0)
    g = lax.dot_general(h, w2g_ref[...], dn, preferred_element_type=jnp.float32)
    gamma_ref[...] = jax.nn.sigmoid(g + b2g_ref[...])
    bb = lax.dot_general(h, w2b_ref[...], dn, preferred_element_type=jnp.float32)
    beta_ref[...] = jax.nn.sigmoid(bb + b2b_ref[...])


def _film_params(pose_p, W1, b1, W2g, b2g, W2b, b2b):
    n_blk = E_PAD // EB_TC
    full = lambda i: (0, 0)
    return pl.pallas_call(
        _film_body,
        grid=(n_blk,),
        in_specs=[
            pl.BlockSpec((EB_TC, 16), lambda i: (i, 0)),
            pl.BlockSpec((C_DIM, 16), full),
            pl.BlockSpec((1, C_DIM), full),
            pl.BlockSpec((C_DIM, C_DIM), full),
            pl.BlockSpec((1, C_DIM), full),
            pl.BlockSpec((C_DIM, C_DIM), full),
            pl.BlockSpec((1, C_DIM), full),
        ],
        out_specs=[
            pl.BlockSpec((EB_TC, C_DIM), lambda i: (i, 0)),
            pl.BlockSpec((EB_TC, C_DIM), lambda i: (i, 0)),
        ],
        out_shape=[
            jax.ShapeDtypeStruct((E_PAD, C_DIM), jnp.float32),
            jax.ShapeDtypeStruct((E_PAD, C_DIM), jnp.float32),
        ],
    )(pose_p, W1, b1, W2g, b2g, W2b, b2b)


def _sc_body(xt_hbm, gamma_hbm, beta_hbm, src_hbm, dst_hbm, z_hbm,
             msum_hbm, cntp_hbm,
             src_v, dst_v, srca_v,
             xb0, xb1, gb0, gb1, bb0, bb1, pb0, pb1,
             isem0, isem1, osem0, osem1, acc_sh):
    cid = lax.axis_index("c")
    sid = lax.axis_index("s")
    e0 = sid * ET
    row0 = sid * ROWS_PER_TILE
    xbuf = (xb0, xb1)
    gbuf = (gb0, gb1)
    bbuf = (bb0, bb1)
    pbuf = (pb0, pb1)
    isem = (isem0, isem1)
    osem = (osem0, osem1)

    pltpu.sync_copy(src_hbm.at[pl.ds(e0, ET)], src_v)
    for j in range(NB):
        pltpu.sync_copy(dst_hbm.at[pl.ds(e0 + j * B, B)], dst_v.at[j])

    def zero_acc(acc_sh):
        pltpu.sync_copy(z_hbm, acc_sh.at[pl.ds(row0, ROWS_PER_TILE)])

    def issue_inputs(b, s, c):
        pltpu.async_copy(xt_hbm.at[srca_v.at[pl.ds(b * B, B)]],
                         xbuf[s], isem[s])
        pltpu.async_copy(gamma_hbm.at[pl.ds(e0 + b * B, B),
                                      pl.ds(c * LANES, LANES)],
                         gbuf[s], isem[s])
        pltpu.async_copy(beta_hbm.at[pl.ds(e0 + b * B, B),
                                     pl.ds(c * LANES, LANES)],
                         bbuf[s], isem[s])

    def wait_inputs(b, s, c):
        pltpu.make_async_copy(xt_hbm.at[srca_v.at[pl.ds(b * B, B)]],
                              xbuf[s], isem[s]).wait()
        pltpu.make_async_copy(gamma_hbm.at[pl.ds(e0 + b * B, B),
                                           pl.ds(c * LANES, LANES)],
                              gbuf[s], isem[s]).wait()
        pltpu.make_async_copy(beta_hbm.at[pl.ds(e0 + b * B, B),
                                          pl.ds(c * LANES, LANES)],
                              bbuf[s], isem[s]).wait()

    def adjust(c):
        def adj(r, _):
            sl = pl.ds(r * 16, 16)
            srca_v[sl] = src_v[sl] + c
            return 0
        lax.fori_loop(0, ET // 16, adj, 0)

    def run():
        zero_acc(acc_sh)
        c0 = cid * CH_PER_CORE
        adjust(c0)
        issue_inputs(0, 0, c0)
        issue_inputs(1, 1, c0)
        plsc.subcore_barrier()

        for j in range(CH_PER_CORE):
            c = cid * CH_PER_CORE + j

            def step(i, _):
                for s in (0, 1):
                    b = 2 * i + s
                    wait_inputs(b, s, c)

                    @pl.when(i > 0)
                    def _():
                        pltpu.make_async_copy(
                            pbuf[s], acc_sh.at[dst_v.at[b]], osem[s]).wait()

                    def mul_row(r, _):
                        for k in range(LANES // 16):
                            sl = pl.ds(k * 16, 16)
                            pbuf[s][r, sl] = (xbuf[s][r, sl] * gbuf[s][r, sl]
                                              + bbuf[s][r, sl])
                        return 0
                    lax.fori_loop(0, B, mul_row, 0)

                    @pl.when(b + 2 < NB)
                    def _():
                        issue_inputs(b + 2, s, c)

                    pltpu.async_copy(pbuf[s], acc_sh.at[dst_v.at[b]],
                                     osem[s], add=True)
                return 0
            lax.fori_loop(0, NB // 2, step, 0)
            for s in (0, 1):
                pltpu.make_async_copy(pbuf[s], acc_sh.at[dst_v.at[0]],
                                      osem[s]).wait()

            if j != CH_PER_CORE - 1:
                adjust(c + 1)
                issue_inputs(0, 0, c + 1)
                issue_inputs(1, 1, c + 1)

            plsc.subcore_barrier()
            pltpu.sync_copy(acc_sh.at[pl.ds(row0, ROWS_PER_TILE)],
                            msum_hbm.at[pl.ds(c * N_PAD + row0,
                                              ROWS_PER_TILE)])
            zero_acc(acc_sh)
            plsc.subcore_barrier()

        @pl.when(cid == 0)
        def _():
            ones16 = jnp.ones((16,), jnp.float32)

            def orow(r, _):
                for k in range(LANES // 16):
                    pb0[r, pl.ds(k * 16, 16)] = ones16
                return 0
            lax.fori_loop(0, B, orow, 0)
            for b in range(NB):
                pltpu.sync_copy(pb0, acc_sh.at[dst_v.at[b]], add=True)
            plsc.subcore_barrier()
            pltpu.sync_copy(acc_sh.at[pl.ds(row0, ROWS_PER_TILE)],
                            cntp_hbm.at[pl.ds(row0, ROWS_PER_TILE)])

    run()


def _sc_aggregate(x_r, gamma, beta, src_p, dst_p, z):
    mesh = plsc.VectorSubcoreMesh(core_axis_name="c", subcore_axis_name="s")
    fbuf = pltpu.VMEM((B, LANES), jnp.float32)
    return pl.kernel(
        _sc_body,
        out_type=[
            jax.ShapeDtypeStruct((NCH * N_PAD, LANES), jnp.float32),
            jax.ShapeDtypeStruct((N_PAD, LANES), jnp.float32),
        ],
        mesh=mesh,
        compiler_params=pltpu.CompilerParams(use_tc_tiling_on_sc=True),
        scratch_types=[
            pltpu.VMEM((ET,), jnp.int32),
            pltpu.VMEM((NB, B), jnp.int32),
            pltpu.VMEM((ET,), jnp.int32),
            fbuf, fbuf, fbuf, fbuf, fbuf, fbuf, fbuf, fbuf,
            pltpu.SemaphoreType.DMA,
            pltpu.SemaphoreType.DMA,
            pltpu.SemaphoreType.DMA,
            pltpu.SemaphoreType.DMA,
            pltpu.VMEM_SHARED((N_PAD, LANES), jnp.float32),
        ],
    )(x_r, gamma, beta, src_p, dst_p, z)


def _div_body(msum_ref, cntp_ref, out_ref):
    cnt = cntp_ref[:, 0]
    inv = 1.0 / jnp.maximum(cnt, 1.0)
    out_ref[...] = msum_ref[...] * inv[:, None]


def _mean_divide(msum, cntp):
    nblk = 2048
    grid = (N_PAD // nblk, NCH)
    return pl.pallas_call(
        _div_body,
        grid=grid,
        in_specs=[
            pl.BlockSpec((nblk, LANES),
                         lambda i, c: (c * (N_PAD // nblk) + i, 0)),
            pl.BlockSpec((nblk, LANES), lambda i, c: (i, 0)),
        ],
        out_specs=pl.BlockSpec((nblk, LANES), lambda i, c: (i, c)),
        out_shape=jax.ShapeDtypeStruct((N_PAD, C_DIM), jnp.float32),
    )(msum, cntp)


@jax.jit
def kernel(x, pose, W1, b1, W2, b2, edge_index):
    x_r = x.reshape(N_NODES * NCH, LANES)

    src = edge_index[0]
    dst = edge_index[1]
    pad = E_PAD - N_EDGES
    src_p = jnp.concatenate([src * NCH, jnp.zeros((pad,), jnp.int32)])
    dst_p = jnp.concatenate([dst, jnp.full((pad,), N_NODES, jnp.int32)])
    pose_p = jnp.zeros((E_PAD, 16), jnp.float32).at[:N_EDGES, :9].set(pose)

    W1p = jnp.zeros((C_DIM, 16), jnp.float32).at[:, :9].set(W1)
    W2g, W2b = W2[0::2], W2[1::2]
    b2g, b2b = b2[0::2], b2[1::2]

    gamma, beta = _film_params(pose_p, W1p, b1.reshape(1, C_DIM),
                               W2g, b2g.reshape(1, C_DIM),
                               W2b, b2b.reshape(1, C_DIM))

    z = jnp.zeros((ROWS_PER_TILE, LANES), jnp.float32)
    msum, cntp = _sc_aggregate(x_r, gamma, beta, src_p, dst_p, z)

    out = _mean_divide(msum, cntp)
    return out[:N_NODES].reshape(N_NODES, C_DIM, 1, 1)
